# baseline (speedup 1.0000x reference)
import functools

import jax
import jax.numpy as jnp
from jax.experimental import pallas as pl
from jax.experimental.pallas import tpu as pltpu

Q = 1024
D = 256
M = 100000
BN = 8192
NB = (M + BN - 1) // BN
KNN = 8
LANES = 128
NCHUNK = BN // LANES
CAND = KNN * LANES
NEG = -jnp.inf


def _start_fetch(m_hbm, mbuf, sems, idx, slot):
    st = jnp.minimum(idx * BN, M - BN)
    pltpu.make_async_copy(m_hbm.at[pl.ds(st, BN), :], mbuf.at[slot],
                          sems.at[slot]).start()


def _knn_body(q_ref, m_hbm, acc_ref, qn_ref, mbuf, sems):
    j = pl.program_id(0)
    slot = jax.lax.rem(j, 2)

    @pl.when(j == 0)
    def _init():
        _start_fetch(m_hbm, mbuf, sems, j, slot)
        acc_ref[...] = jnp.full_like(acc_ref, NEG)
        q = q_ref[...]
        qn = q * jax.lax.rsqrt(
            jnp.maximum(jnp.sum(q * q, axis=1, keepdims=True), 1e-24))
        qn_ref[...] = qn.astype(jnp.bfloat16)

    @pl.when(j + 1 < NB)
    def _prefetch():
        _start_fetch(m_hbm, mbuf, sems, j + 1, 1 - slot)

    st = jnp.minimum(j * BN, M - BN)
    pltpu.make_async_copy(m_hbm.at[pl.ds(st, BN), :], mbuf.at[slot],
                          sems.at[slot]).wait()

    m = mbuf[slot]
    mn = m * jax.lax.rsqrt(
        jnp.maximum(jnp.sum(m * m, axis=1, keepdims=True), 1e-24))
    dup = j * BN - st
    row = jax.lax.broadcasted_iota(jnp.int32, (BN, 1), 0)
    mn = jnp.where(row >= dup, mn, 0.0).astype(jnp.bfloat16)

    def _ce(a, b):
        return jnp.maximum(a, b), jnp.minimum(a, b)

    def _bitonic_sort_desc(seq):
        seq = list(seq)
        n = len(seq)
        d = n // 2
        while d >= 1:
            for start in range(0, n, 2 * d):
                for i in range(start, start + d):
                    seq[i], seq[i + d] = _ce(seq[i], seq[i + d])
            d //= 2
        return seq

    def _merge_desc(a, b):
        return _bitonic_sort_desc(list(a) + list(b)[::-1])

    def _topk_merge(a, b):
        t = [jnp.maximum(a[i], b[KNN - 1 - i]) for i in range(KNN)]
        return _bitonic_sort_desc(t)

    SUB = 2048
    sub_runs = []
    for s in range(BN // SUB):
        ps = jax.lax.dot_general(
            qn_ref[...], mn[s * SUB:(s + 1) * SUB, :],
            (((1,), (1,)), ((), ())),
            preferred_element_type=jnp.float32).astype(jnp.bfloat16)
        chunks = [ps[:, c * LANES:(c + 1) * LANES]
                  for c in range(SUB // LANES)]
        runs = [list(_ce(chunks[2 * k], chunks[2 * k + 1]))
                for k in range(len(chunks) // 2)]
        while len(runs[0]) < KNN:
            runs = [_merge_desc(runs[2 * k], runs[2 * k + 1])
                    for k in range(len(runs) // 2)]
        while len(runs) > 1:
            runs = [_topk_merge(runs[2 * k], runs[2 * k + 1])
                    for k in range(len(runs) // 2)]
        sub_runs.append(runs[0])
    while len(sub_runs) > 1:
        sub_runs = [_topk_merge(sub_runs[2 * k], sub_runs[2 * k + 1])
                    for k in range(len(sub_runs) // 2)]
    accs = [acc_ref[:, i * LANES:(i + 1) * LANES] for i in range(KNN)]
    new_acc = _topk_merge(accs, sub_runs[0])
    for i in range(KNN):
        acc_ref[:, i * LANES:(i + 1) * LANES] = new_acc[i]


def _merge_body(acc_ref, out_ref):
    cand = acc_ref[...].astype(jnp.float32)
    iota = jax.lax.broadcasted_iota(jnp.int32, (Q, CAND), 1)
    vals = []
    work = cand
    for i in range(KNN):
        mx = jnp.max(work, axis=1, keepdims=True)
        vals.append(mx)
        if i < KNN - 1:
            idx = jnp.max(jnp.where(work == mx, iota, -1), axis=1,
                          keepdims=True)
            work = jnp.where(iota == idx, NEG, work)
    top = jnp.concatenate(vals, axis=1)
    out_ref[...] = 1.0 - jnp.mean(top, axis=1, keepdims=True)


@functools.partial(jax.jit, static_argnames=("interpret",))
def kernel(query_embeddings, memory_bank, interpret=False):
    acc = pl.pallas_call(
        _knn_body,
        grid=(NB,),
        in_specs=[
            pl.BlockSpec((Q, D), lambda j: (0, 0)),
            pl.BlockSpec(memory_space=pl.ANY),
        ],
        out_specs=pl.BlockSpec((Q, CAND), lambda j: (0, 0)),
        out_shape=jax.ShapeDtypeStruct((Q, CAND), jnp.bfloat16),
        scratch_shapes=[
            pltpu.VMEM((Q, D), jnp.bfloat16),
            pltpu.VMEM((2, BN, D), jnp.float32),
            pltpu.SemaphoreType.DMA((2,)),
        ],
        compiler_params=pltpu.CompilerParams(
            dimension_semantics=("arbitrary",)),
        interpret=interpret,
    )(query_embeddings, memory_bank)

    out = pl.pallas_call(
        _merge_body,
        grid=(1,),
        in_specs=[pl.BlockSpec((Q, CAND), lambda i: (0, 0))],
        out_specs=pl.BlockSpec((Q, 1), lambda i: (0, 0)),
        out_shape=jax.ShapeDtypeStruct((Q, 1), jnp.float32),
        interpret=interpret,
    )(acc)
    return out.reshape(Q)


# speedup vs baseline: 1.0026x; 1.0026x over previous
import functools

import jax
import jax.numpy as jnp
from jax.experimental import pallas as pl
from jax.experimental.pallas import tpu as pltpu

Q = 1024
D = 256
M = 100000
BN = 4096
NB = (M + BN - 1) // BN
KNN = 8
LANES = 128
NCHUNK = BN // LANES
CAND = KNN * LANES
NEG = -jnp.inf


def _start_fetch(m_hbm, mbuf, sems, idx, slot):
    st = jnp.minimum(idx * BN, M - BN)
    pltpu.make_async_copy(m_hbm.at[pl.ds(st, BN), :], mbuf.at[slot],
                          sems.at[slot]).start()


def _knn_body(q_ref, m_hbm, acc_ref, qn_ref, mbuf, sems):
    j = pl.program_id(0)
    slot = jax.lax.rem(j, 2)

    @pl.when(j == 0)
    def _init():
        _start_fetch(m_hbm, mbuf, sems, j, slot)
        acc_ref[...] = jnp.full_like(acc_ref, NEG)
        q = q_ref[...]
        qn = q * jax.lax.rsqrt(
            jnp.maximum(jnp.sum(q * q, axis=1, keepdims=True), 1e-24))
        qn_ref[...] = qn.astype(jnp.bfloat16)

    @pl.when(j + 1 < NB)
    def _prefetch():
        _start_fetch(m_hbm, mbuf, sems, j + 1, 1 - slot)

    st = jnp.minimum(j * BN, M - BN)
    pltpu.make_async_copy(m_hbm.at[pl.ds(st, BN), :], mbuf.at[slot],
                          sems.at[slot]).wait()

    m = mbuf[slot]
    mn = m * jax.lax.rsqrt(
        jnp.maximum(jnp.sum(m * m, axis=1, keepdims=True), 1e-24))
    dup = j * BN - st
    row = jax.lax.broadcasted_iota(jnp.int32, (BN, 1), 0)
    mn = jnp.where(row >= dup, mn, 0.0).astype(jnp.bfloat16)

    def _ce(a, b):
        return jnp.maximum(a, b), jnp.minimum(a, b)

    def _bitonic_sort_desc(seq):
        seq = list(seq)
        n = len(seq)
        d = n // 2
        while d >= 1:
            for start in range(0, n, 2 * d):
                for i in range(start, start + d):
                    seq[i], seq[i + d] = _ce(seq[i], seq[i + d])
            d //= 2
        return seq

    def _merge_desc(a, b):
        return _bitonic_sort_desc(list(a) + list(b)[::-1])

    def _topk_merge(a, b):
        t = [jnp.maximum(a[i], b[KNN - 1 - i]) for i in range(KNN)]
        return _bitonic_sort_desc(t)

    SUB = 1024
    sub_runs = []
    for s in range(BN // SUB):
        ps = jax.lax.dot_general(
            qn_ref[...], mn[s * SUB:(s + 1) * SUB, :],
            (((1,), (1,)), ((), ())),
            preferred_element_type=jnp.float32).astype(jnp.bfloat16)
        chunks = [ps[:, c * LANES:(c + 1) * LANES]
                  for c in range(SUB // LANES)]
        runs = [list(_ce(chunks[2 * k], chunks[2 * k + 1]))
                for k in range(len(chunks) // 2)]
        while len(runs[0]) < KNN:
            runs = [_merge_desc(runs[2 * k], runs[2 * k + 1])
                    for k in range(len(runs) // 2)]
        while len(runs) > 1:
            runs = [_topk_merge(runs[2 * k], runs[2 * k + 1])
                    for k in range(len(runs) // 2)]
        sub_runs.append(runs[0])
    while len(sub_runs) > 1:
        sub_runs = [_topk_merge(sub_runs[2 * k], sub_runs[2 * k + 1])
                    for k in range(len(sub_runs) // 2)]
    accs = [acc_ref[:, i * LANES:(i + 1) * LANES] for i in range(KNN)]
    new_acc = _topk_merge(accs, sub_runs[0])
    for i in range(KNN):
        acc_ref[:, i * LANES:(i + 1) * LANES] = new_acc[i]


def _merge_body(acc_ref, out_ref):
    cand = acc_ref[...].astype(jnp.float32)
    iota = jax.lax.broadcasted_iota(jnp.int32, (Q, CAND), 1)
    vals = []
    work = cand
    for i in range(KNN):
        mx = jnp.max(work, axis=1, keepdims=True)
        vals.append(mx)
        if i < KNN - 1:
            idx = jnp.max(jnp.where(work == mx, iota, -1), axis=1,
                          keepdims=True)
            work = jnp.where(iota == idx, NEG, work)
    top = jnp.concatenate(vals, axis=1)
    out_ref[...] = 1.0 - jnp.mean(top, axis=1, keepdims=True)


@functools.partial(jax.jit, static_argnames=("interpret",))
def kernel(query_embeddings, memory_bank, interpret=False):
    acc = pl.pallas_call(
        _knn_body,
        grid=(NB,),
        in_specs=[
            pl.BlockSpec((Q, D), lambda j: (0, 0)),
            pl.BlockSpec(memory_space=pl.ANY),
        ],
        out_specs=pl.BlockSpec((Q, CAND), lambda j: (0, 0)),
        out_shape=jax.ShapeDtypeStruct((Q, CAND), jnp.bfloat16),
        scratch_shapes=[
            pltpu.VMEM((Q, D), jnp.bfloat16),
            pltpu.VMEM((2, BN, D), jnp.float32),
            pltpu.SemaphoreType.DMA((2,)),
        ],
        compiler_params=pltpu.CompilerParams(
            dimension_semantics=("arbitrary",)),
        interpret=interpret,
    )(query_embeddings, memory_bank)

    out = pl.pallas_call(
        _merge_body,
        grid=(1,),
        in_specs=[pl.BlockSpec((Q, CAND), lambda i: (0, 0))],
        out_specs=pl.BlockSpec((Q, 1), lambda i: (0, 0)),
        out_shape=jax.ShapeDtypeStruct((Q, 1), jnp.float32),
        interpret=interpret,
    )(acc)
    return out.reshape(Q)
